# baseline (speedup 1.0000x reference)
import functools

import jax
import jax.numpy as jnp
from jax import lax
from jax.experimental import pallas as pl
from jax.experimental.pallas import tpu as pltpu
from jax.experimental.pallas import tpu_sc as plsc

MAX_IDX = 8191
D = 768
SCALE = float(D) ** 0.5
LANES = 16
VPR = D // LANES

NC = 2
NS = 16
NW = NC * NS

B = 4 * 8192
B_PER_W = B // NW
CHUNK = 32
N_CHUNKS = B_PER_W // CHUNK
NBUF = 4
LOOKAHEAD = 2


def _embed_body(x_hbm, table_hbm, out_hbm, idx_v, *scratch):
    bufs = scratch[:NBUF]
    gsems = scratch[NBUF:2 * NBUF]
    ssems = scratch[2 * NBUF:3 * NBUF]

    wid = lax.axis_index("s") * NC + lax.axis_index("c")
    base = wid * B_PER_W

    pltpu.sync_copy(x_hbm.at[wid], idx_v)

    def clamp_body(c, carry):
        for k in range(CHUNK // LANES):
            sl = pl.ds(k * LANES, LANES)
            idx_v[c, sl] = jnp.clip(idx_v[c, sl], 0, MAX_IDX)
        return carry

    lax.fori_loop(0, N_CHUNKS, clamp_body, 0)

    def wait_dma(sem, dst_buf):
        pltpu.make_async_copy(table_hbm.at[pl.ds(0, CHUNK)], dst_buf, sem).wait()

    for j in range(LOOKAHEAD):
        pltpu.async_copy(table_hbm.at[idx_v.at[j]], bufs[j], gsems[j])

    def group_body(g, carry):
        for b in range(NBUF):
            ci = g * NBUF + b
            wait_dma(gsems[b], bufs[b])

            def row_body(r, c2):
                for j in range(VPR):
                    sl = pl.ds(j * LANES, LANES)
                    bufs[b][r, sl] = bufs[b][r, sl] * SCALE
                return c2

            lax.fori_loop(0, CHUNK, row_body, 0)

            pltpu.async_copy(
                bufs[b], out_hbm.at[pl.ds(base + ci * CHUNK, CHUNK)], ssems[b])

            nci = ci + LOOKAHEAD
            nb = (b + LOOKAHEAD) % NBUF

            @pl.when(nci < N_CHUNKS)
            def _():
                @pl.when(nci >= NBUF)
                def _():
                    wait_dma(ssems[nb], bufs[nb])

                pltpu.async_copy(table_hbm.at[idx_v.at[nci]], bufs[nb], gsems[nb])

        return carry

    lax.fori_loop(0, N_CHUNKS // NBUF, group_body, 0)

    for b in range(NBUF):
        wait_dma(ssems[b], bufs[b])


@functools.partial(
    pl.kernel,
    out_type=jax.ShapeDtypeStruct((B, D), jnp.float32),
    mesh=plsc.VectorSubcoreMesh(core_axis_name="c", subcore_axis_name="s"),
    scratch_types=(
        [pltpu.VMEM((N_CHUNKS, CHUNK), jnp.int32)]
        + [pltpu.VMEM((CHUNK, D), jnp.float32) for _ in range(NBUF)]
        + [pltpu.SemaphoreType.DMA for _ in range(2 * NBUF)]
    ),
)
def _embed_kernel(x_hbm, table_hbm, out_hbm, *scratch):
    _embed_body(x_hbm, table_hbm, out_hbm, *scratch)


def kernel(x, table):
    out = _embed_kernel(x.reshape(NW, N_CHUNKS, CHUNK), table)
    return out.reshape(x.shape + (D,))


# speedup vs baseline: 1.0111x; 1.0111x over previous
import functools

import jax
import jax.numpy as jnp
from jax import lax
from jax.experimental import pallas as pl
from jax.experimental.pallas import tpu as pltpu
from jax.experimental.pallas import tpu_sc as plsc

MAX_IDX = 8191
D = 768
SCALE = float(D) ** 0.5
LANES = 16
VPR = D // LANES

NC = 2
NS = 16
NW = NC * NS

B = 4 * 8192
B_PER_W = B // NW
CHUNK = 16
N_CHUNKS = B_PER_W // CHUNK
NBUF = 4
LOOKAHEAD = 3

SPMEM_POS = (1, 3)


def _embed_body(x_hbm, table_hbm, out_hbm, idx_v, shared, *scratch):
    bufs = scratch[:NBUF]
    gsems = scratch[NBUF:2 * NBUF]
    ssems = scratch[2 * NBUF:3 * NBUF]
    xsems = scratch[3 * NBUF:3 * NBUF + 4]
    dsems = scratch[3 * NBUF + 4:]

    sid = lax.axis_index("s")
    wid = sid * NC + lax.axis_index("c")
    base = wid * B_PER_W

    tpr = 8192 // B_PER_W
    pltpu.sync_copy(
        x_hbm.at[wid // tpr, pl.ds((wid % tpr) * B_PER_W, B_PER_W)], idx_v)

    def clamp_body(c, carry):
        for k in range(CHUNK // LANES):
            sl = pl.ds(c * CHUNK + k * LANES, LANES)
            idx_v[sl] = jnp.clip(idx_v[sl], 0, MAX_IDX)
        return carry

    lax.fori_loop(0, N_CHUNKS, clamp_body, 0)

    def slot_ref(ph, bh):
        return shared.at[sid, ph, bh]

    def slot_of(b, p):
        return (p, SPMEM_POS.index(b))

    def wait_gather(b):
        pltpu.make_async_copy(table_hbm.at[pl.ds(0, CHUNK)], bufs[b],
                              gsems[b]).wait()

    def wait_store(b):
        pltpu.make_async_copy(bufs[b], out_hbm.at[pl.ds(0, CHUNK)],
                              ssems[b]).wait()

    def wait_hop(ph, bh):
        pltpu.make_async_copy(table_hbm.at[pl.ds(0, CHUNK)],
                              slot_ref(ph, bh), xsems[2 * ph + bh]).wait()

    def wait_drain(ph, bh):
        pltpu.make_async_copy(slot_ref(ph, bh), out_hbm.at[pl.ds(0, CHUNK)],
                              dsems[2 * ph + bh]).wait()

    for j in range(LOOKAHEAD):
        pltpu.async_copy(table_hbm.at[idx_v.at[pl.ds(j * CHUNK, CHUNK)]],
                         bufs[j], gsems[j])

    def pair_body(gp, carry):
        for p in range(2):
            for b in range(NBUF):
                ci = (2 * gp + p) * NBUF + b
                wait_gather(b)

                def row_body(r, c2):
                    for j in range(VPR):
                        sl = pl.ds(j * LANES, LANES)
                        bufs[b][r, sl] = bufs[b][r, sl] * SCALE
                    return c2

                lax.fori_loop(0, CHUNK, row_body, 0)

                if b in SPMEM_POS:
                    ph, bh = slot_of(b, p)

                    @pl.when(ci >= 2 * NBUF)
                    def _():
                        wait_drain(ph, bh)

                    pltpu.async_copy(bufs[b], slot_ref(ph, bh),
                                     xsems[2 * ph + bh])

                    oph, obh = slot_of(b, 1 - p)

                    @pl.when(ci >= NBUF)
                    def _():
                        pltpu.async_copy(
                            slot_ref(oph, obh),
                            out_hbm.at[pl.ds(base + (ci - NBUF) * CHUNK,
                                             CHUNK)],
                            dsems[2 * oph + obh])
                else:
                    pltpu.async_copy(
                        bufs[b],
                        out_hbm.at[pl.ds(base + ci * CHUNK, CHUNK)],
                        ssems[b])

                nci = ci + LOOKAHEAD
                nb = (b + LOOKAHEAD) % NBUF
                prev_p = p if b != 0 else 1 - p

                @pl.when(nci < N_CHUNKS)
                def _():
                    @pl.when(nci >= NBUF)
                    def _():
                        if nb in SPMEM_POS:
                            wph, wbh = slot_of(nb, prev_p)
                            wait_hop(wph, wbh)
                        else:
                            wait_store(nb)

                    pltpu.async_copy(
                        table_hbm.at[idx_v.at[pl.ds(nci * CHUNK, CHUNK)]],
                        bufs[nb], gsems[nb])

        return carry

    lax.fori_loop(0, N_CHUNKS // (2 * NBUF), pair_body, 0)

    for b in SPMEM_POS:
        ph, bh = slot_of(b, 1)
        wait_hop(ph, bh)
        ci = N_CHUNKS - NBUF + b
        pltpu.async_copy(slot_ref(ph, bh),
                         out_hbm.at[pl.ds(base + ci * CHUNK, CHUNK)],
                         dsems[2 * ph + bh])
    for b in SPMEM_POS:
        for p in range(2):
            ph, bh = slot_of(b, p)
            wait_drain(ph, bh)
    for b in range(NBUF):
        if b not in SPMEM_POS:
            wait_store(b)


@functools.partial(
    pl.kernel,
    out_type=jax.ShapeDtypeStruct((B, D), jnp.float32),
    mesh=plsc.VectorSubcoreMesh(core_axis_name="c", subcore_axis_name="s"),
    scratch_types=(
        [pltpu.VMEM((B_PER_W,), jnp.int32)]
        + [pltpu.VMEM_SHARED((NS, 2, 2, CHUNK, D), jnp.float32)]
        + [pltpu.VMEM((CHUNK, D), jnp.float32) for _ in range(NBUF)]
        + [pltpu.SemaphoreType.DMA for _ in range(2 * NBUF + 8)]
    ),
)
def _embed_kernel(x_hbm, table_hbm, out_hbm, *scratch):
    _embed_body(x_hbm, table_hbm, out_hbm, *scratch)


def kernel(x, table):
    out = _embed_kernel(x, table)
    return out.reshape(x.shape + (D,))


# speedup vs baseline: 1.0270x; 1.0157x over previous
import functools

import jax
import jax.numpy as jnp
from jax import lax
from jax.experimental import pallas as pl
from jax.experimental.pallas import tpu as pltpu
from jax.experimental.pallas import tpu_sc as plsc

MAX_IDX = 8191
D = 768
SCALE = float(D) ** 0.5
LANES = 16
VPR = D // LANES

NC = 2
NS = 16
NW = NC * NS

B = 4 * 8192
B_PER_W = B // NW
CHUNK = 32
N_CHUNKS = B_PER_W // CHUNK
NBUF = 4
LOOKAHEAD = 3


def _embed_body(x_hbm, table_hbm, out_hbm, idx_v, *scratch):
    bufs = scratch[:NBUF]
    gsems = scratch[NBUF:2 * NBUF]
    ssems = scratch[2 * NBUF:3 * NBUF]

    wid = lax.axis_index("s") * NC + lax.axis_index("c")
    base = wid * B_PER_W

    tpr = 8192 // B_PER_W
    pltpu.sync_copy(
        x_hbm.at[wid // tpr, pl.ds((wid % tpr) * B_PER_W, B_PER_W)], idx_v)

    def clamp_body(c, carry):
        for k in range(CHUNK // LANES):
            sl = pl.ds(c * CHUNK + k * LANES, LANES)
            idx_v[sl] = jnp.clip(idx_v[sl], 0, MAX_IDX)
        return carry

    lax.fori_loop(0, N_CHUNKS, clamp_body, 0)

    def wait_dma(sem, dst_buf):
        pltpu.make_async_copy(table_hbm.at[pl.ds(0, CHUNK)], dst_buf, sem).wait()

    for j in range(LOOKAHEAD):
        pltpu.async_copy(table_hbm.at[idx_v.at[pl.ds(j * CHUNK, CHUNK)]], bufs[j], gsems[j])

    def group_body(g, carry):
        for b in range(NBUF):
            ci = g * NBUF + b
            wait_dma(gsems[b], bufs[b])

            def row_body(r, c2):
                for j in range(VPR):
                    sl = pl.ds(j * LANES, LANES)
                    bufs[b][r, sl] = bufs[b][r, sl] * SCALE
                return c2

            lax.fori_loop(0, CHUNK, row_body, 0)

            pltpu.async_copy(
                bufs[b], out_hbm.at[pl.ds(base + ci * CHUNK, CHUNK)], ssems[b])

            nci = ci + LOOKAHEAD
            nb = (b + LOOKAHEAD) % NBUF

            @pl.when(nci < N_CHUNKS)
            def _():
                @pl.when(nci >= NBUF)
                def _():
                    wait_dma(ssems[nb], bufs[nb])

                pltpu.async_copy(
                    table_hbm.at[idx_v.at[pl.ds(nci * CHUNK, CHUNK)]],
                    bufs[nb], gsems[nb])

        return carry

    lax.fori_loop(0, N_CHUNKS // NBUF, group_body, 0)

    for b in range(NBUF):
        wait_dma(ssems[b], bufs[b])


@functools.partial(
    pl.kernel,
    out_type=jax.ShapeDtypeStruct((B, D), jnp.float32),
    mesh=plsc.VectorSubcoreMesh(core_axis_name="c", subcore_axis_name="s"),
    scratch_types=(
        [pltpu.VMEM((B_PER_W,), jnp.int32)]
        + [pltpu.VMEM((CHUNK, D), jnp.float32) for _ in range(NBUF)]
        + [pltpu.SemaphoreType.DMA for _ in range(2 * NBUF)]
    ),
)
def _embed_kernel(x_hbm, table_hbm, out_hbm, *scratch):
    _embed_body(x_hbm, table_hbm, out_hbm, *scratch)


def kernel(x, table):
    out = _embed_kernel(x, table)
    return out.reshape(x.shape + (D,))


# speedup vs baseline: 1.0310x; 1.0039x over previous
import functools

import jax
import jax.numpy as jnp
from jax import lax
from jax.experimental import pallas as pl
from jax.experimental.pallas import tpu as pltpu
from jax.experimental.pallas import tpu_sc as plsc

MAX_IDX = 8191
D = 768
SCALE = float(D) ** 0.5
LANES = 16
VPR = D // LANES

NC = 2
NS = 16
NW = NC * NS

B = 4 * 8192
B_PER_W = B // NW
CHUNK = 32
N_CHUNKS = B_PER_W // CHUNK
NBUF = 4
LOOKAHEAD = 3


def _embed_body(x_hbm, table_hbm, out_hbm, idx_v, *scratch):
    bufs = scratch[:NBUF]
    gsems = scratch[NBUF:2 * NBUF]
    ssems = scratch[2 * NBUF:3 * NBUF]

    wid = lax.axis_index("s") * NC + lax.axis_index("c")
    base = wid * B_PER_W

    tpr = 8192 // B_PER_W
    pltpu.sync_copy(
        x_hbm.at[wid // tpr, pl.ds((wid % tpr) * B_PER_W, B_PER_W)], idx_v)

    def clamp_body(c, carry):
        for k in range(CHUNK // LANES):
            sl = pl.ds(c * CHUNK + k * LANES, LANES)
            idx_v[sl] = jnp.clip(idx_v[sl], 0, MAX_IDX)
        return carry

    def wait_dma(sem, dst_buf):
        pltpu.make_async_copy(table_hbm.at[pl.ds(0, CHUNK)], dst_buf, sem).wait()

    for j in range(LOOKAHEAD):
        clamp_body(j, 0)
        pltpu.async_copy(table_hbm.at[idx_v.at[pl.ds(j * CHUNK, CHUNK)]], bufs[j], gsems[j])
    lax.fori_loop(LOOKAHEAD, N_CHUNKS, clamp_body, 0)

    def group_body(g, carry):
        for b in range(NBUF):
            ci = g * NBUF + b
            wait_dma(gsems[b], bufs[b])

            def row_body(r, c2):
                for j in range(VPR):
                    sl = pl.ds(j * LANES, LANES)
                    bufs[b][r, sl] = bufs[b][r, sl] * SCALE
                return c2

            lax.fori_loop(0, CHUNK, row_body, 0)

            pltpu.async_copy(
                bufs[b], out_hbm.at[pl.ds(base + ci * CHUNK, CHUNK)], ssems[b])

            nci = ci + LOOKAHEAD
            nb = (b + LOOKAHEAD) % NBUF

            @pl.when(nci < N_CHUNKS)
            def _():
                @pl.when(nci >= NBUF)
                def _():
                    wait_dma(ssems[nb], bufs[nb])

                pltpu.async_copy(
                    table_hbm.at[idx_v.at[pl.ds(nci * CHUNK, CHUNK)]],
                    bufs[nb], gsems[nb])

        return carry

    lax.fori_loop(0, N_CHUNKS // NBUF, group_body, 0)

    for b in range(NBUF):
        wait_dma(ssems[b], bufs[b])


@functools.partial(
    pl.kernel,
    out_type=jax.ShapeDtypeStruct((B, D), jnp.float32),
    mesh=plsc.VectorSubcoreMesh(core_axis_name="c", subcore_axis_name="s"),
    scratch_types=(
        [pltpu.VMEM((B_PER_W,), jnp.int32)]
        + [pltpu.VMEM((CHUNK, D), jnp.float32) for _ in range(NBUF)]
        + [pltpu.SemaphoreType.DMA for _ in range(2 * NBUF)]
    ),
)
def _embed_kernel(x_hbm, table_hbm, out_hbm, *scratch):
    _embed_body(x_hbm, table_hbm, out_hbm, *scratch)


def kernel(x, table):
    out = _embed_kernel(x, table)
    return out.reshape(x.shape + (D,))


# speedup vs baseline: 1.0333x; 1.0023x over previous
import functools

import jax
import jax.numpy as jnp
from jax import lax
from jax.experimental import pallas as pl
from jax.experimental.pallas import tpu as pltpu
from jax.experimental.pallas import tpu_sc as plsc

MAX_IDX = 8191
D = 768
SCALE = float(D) ** 0.5
LANES = 16
VPR = D // LANES

NC = 2
NS = 16
NW = NC * NS

B = 4 * 8192
B_PER_W = B // NW
CHUNK = 32
N_CHUNKS = B_PER_W // CHUNK
NBUF = 4
LOOKAHEAD = 3


def _embed_body(x_hbm, table_hbm, out_hbm, idx_v, *scratch):
    bufs = scratch[:NBUF]
    gsems = scratch[NBUF:2 * NBUF]
    ssems = scratch[2 * NBUF:3 * NBUF]

    wid = lax.axis_index("s") * NC + lax.axis_index("c")
    base = wid * B_PER_W

    tpr = 8192 // B_PER_W
    pltpu.sync_copy(
        x_hbm.at[wid // tpr, pl.ds((wid % tpr) * B_PER_W, B_PER_W)], idx_v)

    def clamp_body(c, carry):
        for k in range(CHUNK // LANES):
            sl = pl.ds(c * CHUNK + k * LANES, LANES)
            idx_v[sl] = jnp.clip(idx_v[sl], 0, MAX_IDX)
        return carry

    def wait_dma(sem, dst_buf):
        pltpu.make_async_copy(table_hbm.at[pl.ds(0, CHUNK)], dst_buf, sem).wait()

    for j in range(LOOKAHEAD):
        clamp_body(j, 0)
        pltpu.async_copy(table_hbm.at[idx_v.at[pl.ds(j * CHUNK, CHUNK)]], bufs[j], gsems[j])
    lax.fori_loop(LOOKAHEAD, N_CHUNKS, clamp_body, 0)

    def group_body(g, carry):
        for b in range(NBUF):
            ci = g * NBUF + b
            wait_dma(gsems[b], bufs[b])

            def row_body(r, c2):
                for j in range(VPR):
                    sl = pl.ds(j * LANES, LANES)
                    bufs[b][r, sl] = bufs[b][r, sl] * SCALE
                return c2

            lax.fori_loop(0, CHUNK, row_body, 0)

            half = CHUNK // 2
            pltpu.async_copy(
                bufs[b].at[pl.ds(0, half)],
                out_hbm.at[pl.ds(base + ci * CHUNK, half)], ssems[b])
            pltpu.async_copy(
                bufs[b].at[pl.ds(half, half)],
                out_hbm.at[pl.ds(base + ci * CHUNK + half, half)], ssems[b])

            nci = ci + LOOKAHEAD
            nb = (b + LOOKAHEAD) % NBUF

            @pl.when(nci < N_CHUNKS)
            def _():
                @pl.when(nci >= NBUF)
                def _():
                    wait_dma(ssems[nb], bufs[nb])

                pltpu.async_copy(
                    table_hbm.at[idx_v.at[pl.ds(nci * CHUNK, CHUNK)]],
                    bufs[nb], gsems[nb])

        return carry

    lax.fori_loop(0, N_CHUNKS // NBUF, group_body, 0)

    for b in range(NBUF):
        wait_dma(ssems[b], bufs[b])


@functools.partial(
    pl.kernel,
    out_type=jax.ShapeDtypeStruct((B, D), jnp.float32),
    mesh=plsc.VectorSubcoreMesh(core_axis_name="c", subcore_axis_name="s"),
    scratch_types=(
        [pltpu.VMEM((B_PER_W,), jnp.int32)]
        + [pltpu.VMEM((CHUNK, D), jnp.float32) for _ in range(NBUF)]
        + [pltpu.SemaphoreType.DMA for _ in range(2 * NBUF)]
    ),
)
def _embed_kernel(x_hbm, table_hbm, out_hbm, *scratch):
    _embed_body(x_hbm, table_hbm, out_hbm, *scratch)


def kernel(x, table):
    out = _embed_kernel(x, table)
    return out.reshape(x.shape + (D,))
